# baseline (speedup 1.0000x reference)
import jax
import jax.numpy as jnp
from jax.experimental import pallas as pl
from jax.experimental.pallas import tpu as pltpu

B, S, D = 4, 4096, 2048
N = B * S
ROWS = 512


def _select_body(m_ref, x_ref, a_ref, o_ref):
    m = m_ref[...]
    o_ref[...] = jnp.where(m != 0, a_ref[...], x_ref[...])


def kernel(x, attack, attack_mask):
    xf = x.reshape(N, D)
    af = attack.reshape(N, D)
    mf = attack_mask.reshape(N, 1).astype(jnp.int32)
    grid = N // ROWS
    out = pl.pallas_call(
        _select_body,
        grid=(grid,),
        in_specs=[
            pl.BlockSpec((ROWS, 1), lambda i: (i, 0)),
            pl.BlockSpec((ROWS, D), lambda i: (i, 0)),
            pl.BlockSpec((ROWS, D), lambda i: (i, 0)),
        ],
        out_specs=pl.BlockSpec((ROWS, D), lambda i: (i, 0)),
        out_shape=jax.ShapeDtypeStruct((N, D), jnp.float32),
        compiler_params=pltpu.CompilerParams(
            dimension_semantics=("arbitrary",),
        ),
    )(mf, xf, af)
    return out.reshape(B, S, D)
